# baseline (speedup 1.0000x reference)
import functools

import jax
import jax.numpy as jnp
from jax import lax
from jax.experimental import pallas as pl
from jax.experimental.pallas import tpu as pltpu
from jax.experimental.pallas import tpu_sc as plsc

EMB = 32
GRP = 4
NW = 32
CHUNK = 64


def _pack128(tab):
    return jnp.concatenate([tab[a::GRP] for a in range(GRP)], axis=1)


def _sc_gather_groups(ugidx, mgidx, utab, mtab):
    b = ugidx.shape[0]
    w = b // NW
    nch = w // CHUNK
    d = utab.shape[1]
    mesh = plsc.VectorSubcoreMesh(core_axis_name="core", subcore_axis_name="subcore")

    @functools.partial(
        pl.kernel,
        out_type=(
            jax.ShapeDtypeStruct((b, d), jnp.float32),
            jax.ShapeDtypeStruct((b, d), jnp.float32),
        ),
        mesh=mesh,
        scratch_types=[
            pltpu.VMEM((w,), jnp.int32),
            pltpu.VMEM((w,), jnp.int32),
            pltpu.VMEM((CHUNK, 128), jnp.float32),
            pltpu.VMEM((CHUNK, 128), jnp.float32),
            pltpu.SemaphoreType.DMA,
            pltpu.SemaphoreType.DMA,
            pltpu.SemaphoreType.DMA,
        ],
    )
    def gather_kernel(utab_hbm, mtab_hbm, uidx_hbm, midx_hbm, uout_hbm, mout_hbm,
                      uidx_v, midx_v, buf0, buf1, sem_i, sem0, sem1):
        wid = lax.axis_index("subcore") * 2 + lax.axis_index("core")
        base = wid * w
        cpu = pltpu.async_copy(uidx_hbm.at[pl.ds(base, w)], uidx_v, sem_i)
        cpm = pltpu.async_copy(midx_hbm.at[pl.ds(base, w)], midx_v, sem_i)
        cpu.wait()
        cpm.wait()

        bufs = (buf0, buf1)
        sems = (sem0, sem1)
        for tab_hbm, idx_v, out_hbm in ((utab_hbm, uidx_v, uout_hbm),
                                        (mtab_hbm, midx_v, mout_hbm)):
            cps = [None, None]
            for c in range(nch):
                p = c & 1
                if cps[p] is not None:
                    cps[p].wait()
                cps[p] = pltpu.async_copy(
                    tab_hbm.at[idx_v.at[pl.ds(c * CHUNK, CHUNK)]], bufs[p], sems[p])
                if c > 0:
                    q = 1 - p
                    cps[q].wait()
                    cps[q] = None
                    pltpu.sync_copy(
                        bufs[q], out_hbm.at[pl.ds(base + (c - 1) * CHUNK, CHUNK)])
            p = (nch - 1) & 1
            cps[p].wait()
            pltpu.sync_copy(
                bufs[p], out_hbm.at[pl.ds(base + (nch - 1) * CHUNK, CHUNK)])

    return gather_kernel(utab, mtab, ugidx, mgidx)


def _mlp_body(ug_ref, mg_ref, us_ref, ms_ref, w1u_ref, w1m_ref, b1_ref,
              w2_ref, b2_ref, w3_ref, b3_ref, o_ref):
    dn = (((1,), (1,)), ((), ()))
    hp = jax.lax.Precision.HIGHEST
    ug = ug_ref[...]
    mg = mg_ref[...]
    us = us_ref[...]
    ms = ms_ref[...]
    u = jnp.zeros((ug.shape[0], EMB), jnp.float32)
    m = jnp.zeros((mg.shape[0], EMB), jnp.float32)
    for a in range(GRP):
        u += ug[:, a * EMB:(a + 1) * EMB] * (us == a).astype(jnp.float32)[:, None]
        m += mg[:, a * EMB:(a + 1) * EMB] * (ms == a).astype(jnp.float32)[:, None]
    h = lax.dot_general(u, w1u_ref[...], dn, precision=hp,
                        preferred_element_type=jnp.float32)
    h += lax.dot_general(m, w1m_ref[...], dn, precision=hp,
                         preferred_element_type=jnp.float32)
    h = jnp.maximum(h + b1_ref[...][None, :], 0.0)
    h = lax.dot_general(h, w2_ref[...], dn, precision=hp,
                        preferred_element_type=jnp.float32)
    h = jnp.maximum(h + b2_ref[...][None, :], 0.0)
    o_ref[...] = jnp.sum(h * w3_ref[...][0][None, :], axis=1) + b3_ref[...]


def _tc_mlp(ugrp, mgrp, usub, msub, W1, b1, W2, b2, W3, b3):
    b = ugrp.shape[0]
    bm = 2048
    w1u = W1[:, :EMB]
    w1m = W1[:, EMB:]
    grid = (b // bm,)
    return pl.pallas_call(
        _mlp_body,
        grid=grid,
        in_specs=[
            pl.BlockSpec((bm, 128), lambda i: (i, 0)),
            pl.BlockSpec((bm, 128), lambda i: (i, 0)),
            pl.BlockSpec((bm,), lambda i: (i,)),
            pl.BlockSpec((bm,), lambda i: (i,)),
            pl.BlockSpec(w1u.shape, lambda i: (0, 0)),
            pl.BlockSpec(w1m.shape, lambda i: (0, 0)),
            pl.BlockSpec(b1.shape, lambda i: (0,)),
            pl.BlockSpec(W2.shape, lambda i: (0, 0)),
            pl.BlockSpec(b2.shape, lambda i: (0,)),
            pl.BlockSpec(W3.shape, lambda i: (0, 0)),
            pl.BlockSpec(b3.shape, lambda i: (0,)),
        ],
        out_specs=pl.BlockSpec((bm,), lambda i: (i,)),
        out_shape=jax.ShapeDtypeStruct((b,), jnp.float32),
    )(ugrp, mgrp, usub, msub, w1u, w1m, b1, W2, b2, W3, b3)


def kernel(user_idx, movie_idx, user_table, movie_table, W1, b1, W2, b2, W3, b3):
    uidx = user_idx.astype(jnp.int32)
    midx = movie_idx.astype(jnp.int32)
    u128 = _pack128(user_table)
    m128 = _pack128(movie_table)
    ugrp, mgrp = _sc_gather_groups(uidx >> 2, midx >> 2, u128, m128)
    return _tc_mlp(ugrp, mgrp, uidx & 3, midx & 3, W1, b1, W2, b2, W3, b3)


# speedup vs baseline: 13.6421x; 13.6421x over previous
import functools

import jax
import jax.numpy as jnp
from jax import lax
from jax.experimental import pallas as pl
from jax.experimental.pallas import tpu as pltpu
from jax.experimental.pallas import tpu_sc as plsc

EMB = 32
NW = 32


def _sc_gather_rows(idx, tab, tag):
    b = idx.shape[0]
    w = b // NW
    c_rows = 128
    nch = w // c_rows
    mesh = plsc.VectorSubcoreMesh(core_axis_name="core", subcore_axis_name="subcore")

    @functools.partial(
        pl.kernel,
        out_type=jax.ShapeDtypeStruct((b, EMB), jnp.float32),
        mesh=mesh,
        name=f"gather_{tag}",
        scratch_types=[
            pltpu.VMEM((w,), jnp.int32),
            pltpu.VMEM((c_rows, EMB), jnp.float32),
            pltpu.VMEM((c_rows, EMB), jnp.float32),
            pltpu.SemaphoreType.DMA,
            pltpu.SemaphoreType.DMA,
            pltpu.SemaphoreType.DMA,
            pltpu.SemaphoreType.DMA,
        ],
    )
    def gather_kernel(tab_hbm, idx_hbm, out_hbm,
                      idx_v, buf0, buf1, sem_i, sem0, sem1, sem_w):
        wid = lax.axis_index("subcore") * 2 + lax.axis_index("core")
        base = wid * w
        pltpu.async_copy(idx_hbm.at[pl.ds(base, w)], idx_v, sem_i).wait()

        bufs = (buf0, buf1)
        sems = (sem0, sem1)
        wbs = [None, None]
        for c in range(nch):
            p = c & 1
            if wbs[p] is not None:
                wbs[p].wait()
                wbs[p] = None
            buf = bufs[p]

            @pl.loop(0, c_rows, step=16)
            def _(cc, _c=c, _buf=buf, _sem=sems[p]):
                vec = idx_v[pl.ds(_c * c_rows + cc, 16)]
                for j in range(16):
                    i = vec[j]
                    pltpu.async_copy(tab_hbm.at[pl.ds(i, 1)],
                                     _buf.at[pl.ds(cc + j, 1)], _sem)

            pltpu.make_async_copy(
                out_hbm.at[pl.ds(base + c * c_rows, c_rows)], buf, sems[p]
            ).wait()
            wbs[p] = pltpu.async_copy(
                buf, out_hbm.at[pl.ds(base + c * c_rows, c_rows)], sem_w)
        for wb in wbs:
            if wb is not None:
                wb.wait()

    return gather_kernel(tab, idx)


def _mlp_body(u_ref, m_ref, w1u_ref, w1m_ref, b1_ref,
              w2_ref, b2_ref, w3_ref, b3_ref, o_ref):
    dn = (((1,), (1,)), ((), ()))
    u = u_ref[...]
    m = m_ref[...]
    h = lax.dot_general(u, w1u_ref[...], dn,
                        preferred_element_type=jnp.float32)
    h += lax.dot_general(m, w1m_ref[...], dn,
                         preferred_element_type=jnp.float32)
    h = jnp.maximum(h + b1_ref[...][None, :], 0.0)
    h = lax.dot_general(h, w2_ref[...], dn,
                        preferred_element_type=jnp.float32)
    h = jnp.maximum(h + b2_ref[...][None, :], 0.0)
    o_ref[...] = jnp.sum(h * w3_ref[...][0][None, :], axis=1) + b3_ref[...]


def _tc_mlp(u_vec, m_vec, W1, b1, W2, b2, W3, b3):
    b = u_vec.shape[0]
    bm = 2048
    w1u = W1[:, :EMB]
    w1m = W1[:, EMB:]
    grid = (b // bm,)
    return pl.pallas_call(
        _mlp_body,
        grid=grid,
        in_specs=[
            pl.BlockSpec((bm, EMB), lambda i: (i, 0)),
            pl.BlockSpec((bm, EMB), lambda i: (i, 0)),
            pl.BlockSpec(w1u.shape, lambda i: (0, 0)),
            pl.BlockSpec(w1m.shape, lambda i: (0, 0)),
            pl.BlockSpec(b1.shape, lambda i: (0,)),
            pl.BlockSpec(W2.shape, lambda i: (0, 0)),
            pl.BlockSpec(b2.shape, lambda i: (0,)),
            pl.BlockSpec(W3.shape, lambda i: (0, 0)),
            pl.BlockSpec(b3.shape, lambda i: (0,)),
        ],
        out_specs=pl.BlockSpec((bm,), lambda i: (i,)),
        out_shape=jax.ShapeDtypeStruct((b,), jnp.float32),
    )(u_vec, m_vec, w1u, w1m, b1, W2, b2, W3, b3)


def kernel(user_idx, movie_idx, user_table, movie_table, W1, b1, W2, b2, W3, b3):
    uidx = user_idx.astype(jnp.int32)
    midx = movie_idx.astype(jnp.int32)
    m_vec = _sc_gather_rows(midx, movie_table, "movie")
    u_vec = _sc_gather_rows(uidx, user_table, "user")
    return _tc_mlp(u_vec, m_vec, W1, b1, W2, b2, W3, b3)
